# baseline (speedup 1.0000x reference)
import jax
import jax.numpy as jnp
from jax import lax
from jax.experimental import pallas as pl
from jax.experimental.pallas import tpu as pltpu
from jax.experimental.pallas import tpu_sc as plsc

NUM_CORES = 2
NUM_SUBCORES = 16
LANES = 16
NW = NUM_CORES * NUM_SUBCORES

_BATCH = 16384
_D = 32
_BPW = _BATCH // NW
_WAVE = 8
_NWAVE = _BPW // _WAVE


def _fm_body(uet, iet, ubt, ibt, user, item, gb128, out,
             idx_u, idx_i, u_slab, i_slab, ub_slab, ib_slab, out_v, gbv,
             sem, semb):
    wid = lax.axis_index("s") * NUM_CORES + lax.axis_index("c")
    base = wid * _BPW

    for j in range(4):
        pltpu.sync_copy(user.at[pl.ds(base + j * 128, 128)],
                        idx_u.at[pl.ds(j * 128, 128)])
        pltpu.sync_copy(item.at[pl.ds(base + j * 128, 128)],
                        idx_i.at[pl.ds(j * 128, 128)])
    pltpu.sync_copy(gb128, gbv)
    gbs = gbv[pl.ds(0, LANES)][0]

    d_lo = lax.iota(jnp.int32, LANES)
    d_hi = d_lo + LANES

    def wave(w, carry):
        uv = idx_u[pl.ds(w * _WAVE, LANES)]
        iv = idx_i[pl.ds(w * _WAVE, LANES)]
        cps = []
        for l in range(_WAVE):
            u = uv[l]
            it = iv[l]
            uoff = pl.multiple_of((u >> 7) << 7, 128)
            ioff = pl.multiple_of((it >> 7) << 7, 128)
            cps.append(pltpu.async_copy(uet.at[:, pl.ds(uoff, 128)],
                                        u_slab.at[l], sem))
            cps.append(pltpu.async_copy(iet.at[:, pl.ds(ioff, 128)],
                                        i_slab.at[l], sem))
            cps.append(pltpu.async_copy(ubt.at[:, pl.ds(uoff, 128)],
                                        ub_slab.at[l], semb))
            cps.append(pltpu.async_copy(ibt.at[:, pl.ds(ioff, 128)],
                                        ib_slab.at[l], semb))
        for cp in cps:
            cp.wait()
        acc = jnp.zeros((LANES,), jnp.float32)
        for l in range(_WAVE):
            uc = jnp.full((LANES,), uv[l] & 127, jnp.int32)
            ic = jnp.full((LANES,), iv[l] & 127, jnp.int32)
            ll = jnp.full((LANES,), l, jnp.int32)
            zz = jnp.zeros((LANES,), jnp.int32)
            u0 = plsc.load_gather(u_slab, [ll, d_lo, uc])
            u1 = plsc.load_gather(u_slab, [ll, d_hi, uc])
            i0 = plsc.load_gather(i_slab, [ll, d_lo, ic])
            i1 = plsc.load_gather(i_slab, [ll, d_hi, ic])
            ub = plsc.load_gather(ub_slab, [ll, zz, uc])
            ib = plsc.load_gather(ib_slab, [ll, zz, ic])
            s = jnp.sum(u0 * i0 + u1 * i1) + ub[0] + ib[0] + gbs
            acc = jnp.where(d_lo == l, s, acc)
        out_v[pl.ds(w * _WAVE, LANES)] = acc
        return carry

    lax.fori_loop(0, _NWAVE, wave, 0)
    pltpu.sync_copy(out_v.at[pl.ds(0, _BPW)], out.at[pl.ds(base, _BPW)])


def kernel(user, item, user_embed, item_embed, user_bias, item_bias, global_bias):
    mesh = plsc.VectorSubcoreMesh(core_axis_name="c", subcore_axis_name="s")
    fm = pl.kernel(
        _fm_body,
        out_type=jax.ShapeDtypeStruct((_BATCH,), jnp.float32),
        mesh=mesh,
        scratch_types=[
            pltpu.VMEM((_BPW + LANES,), jnp.int32),
            pltpu.VMEM((_BPW + LANES,), jnp.int32),
            pltpu.VMEM((_WAVE, _D, 128), jnp.float32),
            pltpu.VMEM((_WAVE, _D, 128), jnp.float32),
            pltpu.VMEM((_WAVE, 1, 128), jnp.float32),
            pltpu.VMEM((_WAVE, 1, 128), jnp.float32),
            pltpu.VMEM((_BPW + LANES,), jnp.float32),
            pltpu.VMEM((128,), jnp.float32),
            pltpu.SemaphoreType.DMA,
            pltpu.SemaphoreType.DMA,
        ],
        compiler_params=pltpu.CompilerParams(
            needs_layout_passes=False, use_tc_tiling_on_sc=True,
            disable_bounds_checks=True),
    )
    gb128 = jnp.broadcast_to(global_bias, (128,))
    return fm(user_embed.T, item_embed.T, user_bias.T, item_bias.T,
              user, item, gb128)
